# baseline (speedup 1.0000x reference)
import functools

import jax
import jax.numpy as jnp
import numpy as np
from jax import lax
from jax.experimental import pallas as pl
from jax.experimental.pallas import tpu as pltpu
from jax.experimental.pallas import tpu_sc as plsc

TOPK = 8
WIN = 25
T = 512
NFFT = 1024
K = NFFT // 2 + 1
D = 64
BLOCK = 64
TC_BLOCK = 256
N_SC = 1280


def _constants():
    pad = WIN // 2
    M = np.zeros((T, T), dtype=np.float64)
    for tau in range(T):
        for j in range(-pad, pad + 1):
            src = min(max(tau + j, 0), T - 1)
            M[src, tau] += 1.0 / WIN
    A = np.eye(T, dtype=np.float64) - M

    t = np.arange(T, dtype=np.int64)[:, None]
    k = np.arange(K, dtype=np.int64)[None, :]
    ang = 2.0 * np.pi * ((t * k) % NFFT).astype(np.float64) / NFFT
    C = np.cos(ang)
    S = np.sin(ang)

    kk = np.arange(K, dtype=np.int64)[:, None]
    tt = np.arange(T, dtype=np.int64)[None, :]
    ang2 = 2.0 * np.pi * ((kk * tt) % NFFT).astype(np.float64) / NFFT
    w = np.full((K, 1), 2.0 / NFFT, dtype=np.float64)
    w[0, 0] = 1.0 / NFFT
    w[K - 1, 0] = 1.0 / NFFT
    Ci = np.cos(ang2) * w

    f32 = lambda a: jnp.asarray(a, dtype=jnp.float32)
    return f32(A), f32(C), f32(S), f32(Ci)


def _spectral_topk(x, a_ref, c_ref, s_ref, ci_ref, out_ref):
    dot = functools.partial(
        jax.lax.dot,
        precision=jax.lax.Precision.HIGHEST,
        preferred_element_type=jnp.float32,
    )
    d = dot(x, a_ref[...])
    re = dot(d, c_ref[...])
    im = dot(d, s_ref[...])
    p = re * re + im * im
    ac = dot(p, ci_ref[...])

    lane = jax.lax.broadcasted_iota(jnp.int32, ac.shape, 1)
    ac = jnp.where(lane == 0, jnp.float32(-1e9), ac)

    b = ac.shape[0]
    out_lane = jax.lax.broadcasted_iota(jnp.int32, (b, TOPK), 1)
    out = jnp.zeros((b, TOPK), dtype=jnp.int32)
    work = ac
    for kth in range(TOPK):
        m = jnp.max(work, axis=1, keepdims=True)
        arg = jnp.min(
            jnp.where(work == m, lane, jnp.int32(T)), axis=1, keepdims=True
        )
        out = jnp.where(out_lane == kth, jnp.broadcast_to(arg, (b, TOPK)), out)
        work = jnp.where(lane == arg, jnp.float32(-3e38), work)
    out_ref[...] = out


def _fused_body(x_ref, a_ref, c_ref, s_ref, ci_ref, out_ref):
    x = jnp.mean(x_ref[...], axis=1)
    _spectral_topk(x, a_ref, c_ref, s_ref, ci_ref, out_ref)


def _backend_body(x_ref, a_ref, c_ref, s_ref, ci_ref, out_ref):
    _spectral_topk(x_ref[...], a_ref, c_ref, s_ref, ci_ref, out_ref)


_CONST_SPECS = [
    pl.BlockSpec((T, T), lambda i: (0, 0)),
    pl.BlockSpec((T, K), lambda i: (0, 0)),
    pl.BlockSpec((T, K), lambda i: (0, 0)),
    pl.BlockSpec((K, T), lambda i: (0, 0)),
]


def _tc_fused(Xt, consts, row0):
    n = Xt.shape[0] - row0
    off = row0 // BLOCK
    return pl.pallas_call(
        _fused_body,
        grid=(n // BLOCK,),
        in_specs=[pl.BlockSpec((BLOCK, D, T), lambda i: (i + off, 0, 0))]
        + _CONST_SPECS,
        out_specs=pl.BlockSpec((BLOCK, TOPK), lambda i: (i, 0)),
        out_shape=jax.ShapeDtypeStruct((n, TOPK), jnp.int32),
    )(Xt, *consts)


def _tc_backend(xbar, consts):
    n = xbar.shape[0]
    return pl.pallas_call(
        _backend_body,
        grid=(n // TC_BLOCK,),
        in_specs=[pl.BlockSpec((TC_BLOCK, T), lambda i: (i, 0))]
        + _CONST_SPECS,
        out_specs=pl.BlockSpec((TC_BLOCK, TOPK), lambda i: (i, 0)),
        out_shape=jax.ShapeDtypeStruct((n, TOPK), jnp.int32),
    )(xbar, *consts)


def _sc_mean(Xt, n_rows):
    info = plsc.get_sparse_core_info()
    nw = info.num_cores * info.num_subcores
    rows_per_w = n_rows // nw
    mesh = plsc.VectorSubcoreMesh(core_axis_name="c", subcore_axis_name="s")
    NBUF = 3
    NACC = 8

    @functools.partial(
        pl.kernel,
        out_type=jax.ShapeDtypeStruct((n_rows, T), jnp.float32),
        mesh=mesh,
        scratch_types=[
            pltpu.VMEM((D, T), jnp.float32),
            pltpu.VMEM((D, T), jnp.float32),
            pltpu.VMEM((D, T), jnp.float32),
            pltpu.VMEM((8, T), jnp.float32),
            pltpu.SemaphoreType.DMA,
            pltpu.SemaphoreType.DMA,
            pltpu.SemaphoreType.DMA,
        ],
        compiler_params=pltpu.CompilerParams(
            use_tc_tiling_on_sc=True, needs_layout_passes=False
        ),
    )
    def sc_kernel(x_hbm, out_hbm, b0, b1, b2, obuf, s0, s1, s2):
        wid = lax.axis_index("s") * info.num_cores + lax.axis_index("c")
        base = wid * rows_per_w
        bufs = (b0, b1, b2)
        sems = (s0, s1, s2)

        def reduce_row(buf, j):
            def chunk(c, _):
                accs = [
                    buf[dd, pl.ds(c * 16, 16)] for dd in range(NACC)
                ]
                for dd in range(NACC, D):
                    accs[dd % NACC] = accs[dd % NACC] + buf[
                        dd, pl.ds(c * 16, 16)
                    ]
                while len(accs) > 1:
                    accs = [
                        accs[z] + accs[z + 1] for z in range(0, len(accs), 2)
                    ]
                obuf[j, pl.ds(c * 16, 16)] = accs[0] * jnp.float32(1.0 / D)
                return 0

            lax.fori_loop(0, T // 16, chunk, 0)

        for q in range(NBUF):
            pltpu.async_copy(x_hbm.at[base + q], bufs[q], sems[q])

        def row_step(i, _):
            r = base + i
            j = lax.rem(i, 8)
            for q in range(NBUF):

                @pl.when(lax.rem(i, NBUF) == q)
                def _():
                    pltpu.make_async_copy(
                        x_hbm.at[r], bufs[q], sems[q]
                    ).wait()
                    reduce_row(bufs[q], j)

                    @pl.when(i + NBUF < rows_per_w)
                    def _():
                        pltpu.async_copy(
                            x_hbm.at[r + NBUF], bufs[q], sems[q]
                        )

            @pl.when(j == 7)
            def _():
                start = pl.multiple_of(r - 7, 8)
                pltpu.sync_copy(obuf, out_hbm.at[pl.ds(start, 8)])

            return 0

        lax.fori_loop(0, rows_per_w, row_step, 0)

    return sc_kernel(Xt)


def kernel(X):
    consts = _constants()
    Xt = jnp.transpose(X, (0, 2, 1))
    return _tc_fused(Xt, consts, 0)


# speedup vs baseline: 1.7481x; 1.7481x over previous
import functools

import jax
import jax.numpy as jnp
import numpy as np
from jax import lax
from jax.experimental import pallas as pl
from jax.experimental.pallas import tpu as pltpu
from jax.experimental.pallas import tpu_sc as plsc

TOPK = 8
WIN = 25
T = 512
NFFT = 1024
K = NFFT // 2 + 1
D = 64
BLOCK = 128
TC_BLOCK = 256
N_SC = 1280


def _constants():
    pad = WIN // 2
    M = np.zeros((T, T), dtype=np.float64)
    for tau in range(T):
        for j in range(-pad, pad + 1):
            src = min(max(tau + j, 0), T - 1)
            M[src, tau] += 1.0 / WIN
    A = np.eye(T, dtype=np.float64) - M

    t = np.arange(T, dtype=np.int64)[:, None]
    k = np.arange(K, dtype=np.int64)[None, :]
    ang = 2.0 * np.pi * ((t * k) % NFFT).astype(np.float64) / NFFT
    C = np.cos(ang)
    S = np.sin(ang)

    kk = np.arange(K, dtype=np.int64)[:, None]
    tt = np.arange(T, dtype=np.int64)[None, :]
    ang2 = 2.0 * np.pi * ((kk * tt) % NFFT).astype(np.float64) / NFFT
    w = np.full((K, 1), 2.0 / NFFT, dtype=np.float64)
    w[0, 0] = 1.0 / NFFT
    w[K - 1, 0] = 1.0 / NFFT
    Ci = np.cos(ang2) * w

    AC = A @ C
    AS = A @ S

    f32 = lambda a: jnp.asarray(a, dtype=jnp.float32)
    return f32(AC), f32(AS), f32(Ci)


def _spectral_topk(x, c_ref, s_ref, ci_ref, out_ref):
    dot = functools.partial(
        jax.lax.dot,
        precision=jax.lax.Precision.HIGHEST,
        preferred_element_type=jnp.float32,
    )
    re = dot(x, c_ref[...])
    im = dot(x, s_ref[...])
    p = re * re + im * im
    ac = dot(p, ci_ref[...])

    lane = jax.lax.broadcasted_iota(jnp.int32, ac.shape, 1)
    ac = jnp.where(lane == 0, jnp.float32(-1e9), ac)

    b = ac.shape[0]
    out_lane = jax.lax.broadcasted_iota(jnp.int32, (b, TOPK), 1)
    out = jnp.zeros((b, TOPK), dtype=jnp.int32)
    work = ac
    for kth in range(TOPK):
        m = jnp.max(work, axis=1, keepdims=True)
        arg = jnp.min(
            jnp.where(work == m, lane, jnp.int32(T)), axis=1, keepdims=True
        )
        out = jnp.where(out_lane == kth, jnp.broadcast_to(arg, (b, TOPK)), out)
        work = jnp.where(lane == arg, jnp.float32(-3e38), work)
    out_ref[...] = out


def _fused_body(x_ref, c_ref, s_ref, ci_ref, out_ref):
    x = jnp.mean(x_ref[...], axis=1)
    _spectral_topk(x, c_ref, s_ref, ci_ref, out_ref)


def _backend_body(x_ref, c_ref, s_ref, ci_ref, out_ref):
    _spectral_topk(x_ref[...], c_ref, s_ref, ci_ref, out_ref)


_CONST_SPECS = [
    pl.BlockSpec((T, K), lambda i: (0, 0)),
    pl.BlockSpec((T, K), lambda i: (0, 0)),
    pl.BlockSpec((K, T), lambda i: (0, 0)),
]


def _tc_fused(Xt, consts, row0):
    n = Xt.shape[0] - row0
    off = row0 // BLOCK
    return pl.pallas_call(
        _fused_body,
        grid=(n // BLOCK,),
        in_specs=[pl.BlockSpec((BLOCK, D, T), lambda i: (i + off, 0, 0))]
        + _CONST_SPECS,
        out_specs=pl.BlockSpec((BLOCK, TOPK), lambda i: (i, 0)),
        out_shape=jax.ShapeDtypeStruct((n, TOPK), jnp.int32),
    )(Xt, *consts)


def _tc_backend(xbar, consts):
    n = xbar.shape[0]
    return pl.pallas_call(
        _backend_body,
        grid=(n // TC_BLOCK,),
        in_specs=[pl.BlockSpec((TC_BLOCK, T), lambda i: (i, 0))]
        + _CONST_SPECS,
        out_specs=pl.BlockSpec((TC_BLOCK, TOPK), lambda i: (i, 0)),
        out_shape=jax.ShapeDtypeStruct((n, TOPK), jnp.int32),
    )(xbar, *consts)


def _sc_mean(Xt, n_rows):
    info = plsc.get_sparse_core_info()
    nw = info.num_cores * info.num_subcores
    rows_per_w = n_rows // nw
    mesh = plsc.VectorSubcoreMesh(core_axis_name="c", subcore_axis_name="s")
    NBUF = 3
    NACC = 8

    @functools.partial(
        pl.kernel,
        out_type=jax.ShapeDtypeStruct((n_rows, T), jnp.float32),
        mesh=mesh,
        scratch_types=[
            pltpu.VMEM((D, T), jnp.float32),
            pltpu.VMEM((D, T), jnp.float32),
            pltpu.VMEM((D, T), jnp.float32),
            pltpu.VMEM((8, T), jnp.float32),
            pltpu.SemaphoreType.DMA,
            pltpu.SemaphoreType.DMA,
            pltpu.SemaphoreType.DMA,
        ],
        compiler_params=pltpu.CompilerParams(
            use_tc_tiling_on_sc=True, needs_layout_passes=False
        ),
    )
    def sc_kernel(x_hbm, out_hbm, b0, b1, b2, obuf, s0, s1, s2):
        wid = lax.axis_index("s") * info.num_cores + lax.axis_index("c")
        base = wid * rows_per_w
        bufs = (b0, b1, b2)
        sems = (s0, s1, s2)

        def reduce_row(buf, j):
            def chunk(c, _):
                accs = [
                    buf[dd, pl.ds(c * 16, 16)] for dd in range(NACC)
                ]
                for dd in range(NACC, D):
                    accs[dd % NACC] = accs[dd % NACC] + buf[
                        dd, pl.ds(c * 16, 16)
                    ]
                while len(accs) > 1:
                    accs = [
                        accs[z] + accs[z + 1] for z in range(0, len(accs), 2)
                    ]
                obuf[j, pl.ds(c * 16, 16)] = accs[0] * jnp.float32(1.0 / D)
                return 0

            lax.fori_loop(0, T // 16, chunk, 0)

        for q in range(NBUF):
            pltpu.async_copy(x_hbm.at[base + q], bufs[q], sems[q])

        def row_step(i, _):
            r = base + i
            j = lax.rem(i, 8)
            for q in range(NBUF):

                @pl.when(lax.rem(i, NBUF) == q)
                def _():
                    pltpu.make_async_copy(
                        x_hbm.at[r], bufs[q], sems[q]
                    ).wait()
                    reduce_row(bufs[q], j)

                    @pl.when(i + NBUF < rows_per_w)
                    def _():
                        pltpu.async_copy(
                            x_hbm.at[r + NBUF], bufs[q], sems[q]
                        )

            @pl.when(j == 7)
            def _():
                start = pl.multiple_of(r - 7, 8)
                pltpu.sync_copy(obuf, out_hbm.at[pl.ds(start, 8)])

            return 0

        lax.fori_loop(0, rows_per_w, row_step, 0)

    return sc_kernel(Xt)


def kernel(X):
    consts = _constants()
    Xt = jnp.transpose(X, (0, 2, 1))
    return _tc_fused(Xt, consts, 0)


# speedup vs baseline: 1.7508x; 1.0016x over previous
import functools

import jax
import jax.numpy as jnp
import numpy as np
from jax.experimental import pallas as pl

TOPK = 8
WIN = 25
T = 512
NFFT = 1024
K = NFFT // 2 + 1
D = 64
BLOCK = 128


def _constants():
    pad = WIN // 2
    M = np.zeros((T, T), dtype=np.float64)
    for tau in range(T):
        for j in range(-pad, pad + 1):
            src = min(max(tau + j, 0), T - 1)
            M[src, tau] += 1.0 / WIN
    A = np.eye(T, dtype=np.float64) - M

    t = np.arange(T, dtype=np.int64)[:, None]
    k = np.arange(K, dtype=np.int64)[None, :]
    ang = 2.0 * np.pi * ((t * k) % NFFT).astype(np.float64) / NFFT
    C = np.cos(ang)
    S = np.sin(ang)

    kk = np.arange(K, dtype=np.int64)[:, None]
    tt = np.arange(T, dtype=np.int64)[None, :]
    ang2 = 2.0 * np.pi * ((kk * tt) % NFFT).astype(np.float64) / NFFT
    w = np.full((K, 1), 2.0 / NFFT, dtype=np.float64)
    w[0, 0] = 1.0 / NFFT
    w[K - 1, 0] = 1.0 / NFFT
    Ci = np.cos(ang2) * w

    AC = A @ C
    AS = A @ S

    f32 = lambda a: jnp.asarray(a, dtype=jnp.float32)
    return f32(AC), f32(AS), f32(Ci)


def _fused_body(x_ref, c_ref, s_ref, ci_ref, out_ref):
    dot = functools.partial(
        jax.lax.dot,
        precision=jax.lax.Precision.HIGHEST,
        preferred_element_type=jnp.float32,
    )
    x = jnp.mean(x_ref[...], axis=1)
    re = dot(x, c_ref[...])
    im = dot(x, s_ref[...])
    p = re * re + im * im
    ac = dot(p, ci_ref[...])

    lane = jax.lax.broadcasted_iota(jnp.int32, ac.shape, 1)
    ac = jnp.where(lane == 0, jnp.float32(-1e9), ac)

    b = ac.shape[0]
    out_lane = jax.lax.broadcasted_iota(jnp.int32, (b, TOPK), 1)
    out = jnp.zeros((b, TOPK), dtype=jnp.int32)
    work = ac
    for kth in range(TOPK):
        m = jnp.max(work, axis=1, keepdims=True)
        arg = jnp.min(
            jnp.where(work == m, lane, jnp.int32(T)), axis=1, keepdims=True
        )
        out = jnp.where(out_lane == kth, jnp.broadcast_to(arg, (b, TOPK)), out)
        work = jnp.where(lane == arg, jnp.float32(-3e38), work)
    out_ref[...] = out


def kernel(X):
    BN = X.shape[0]
    consts = _constants()
    Xt = jnp.transpose(X, (0, 2, 1))
    return pl.pallas_call(
        _fused_body,
        grid=(BN // BLOCK,),
        in_specs=[
            pl.BlockSpec((BLOCK, D, T), lambda i: (i, 0, 0)),
            pl.BlockSpec((T, K), lambda i: (0, 0)),
            pl.BlockSpec((T, K), lambda i: (0, 0)),
            pl.BlockSpec((K, T), lambda i: (0, 0)),
        ],
        out_specs=pl.BlockSpec((BLOCK, TOPK), lambda i: (i, 0)),
        out_shape=jax.ShapeDtypeStruct((BN, TOPK), jnp.int32),
    )(Xt, *consts)
